# baseline (speedup 1.0000x reference)
import functools

import jax
import jax.numpy as jnp
from jax import lax
from jax.experimental import pallas as pl
from jax.experimental.pallas import tpu as pltpu
from jax.experimental.pallas import tpu_sc as plsc

_D = 64
_V = 1000000
_B = 4096
_S = 200
_L = 16

_info = plsc.get_sparse_core_info()
_NC, _NS = _info.num_cores, _info.num_subcores
_NW = _NC * _NS
_VT = (_V + 127) // 128
_VT_FULL = _V // 128
_TPW = _VT_FULL // _NW
_VT_REST = _VT_FULL - _TPW * _NW
_V_MAIN = _VT_FULL * 128
_V_TAIL = _V - _V_MAIN

_mesh = plsc.VectorSubcoreMesh(core_axis_name="c", subcore_axis_name="s")


def _iota16(off=0):
  return lax.iota(jnp.int32, 16) + off


def _bfly16(regs):
  for s in (1, 2, 4, 8):
    m = (lax.iota(jnp.int32, 16) & s) == 0
    p = lax.iota(jnp.int32, 16) ^ s
    nr = list(regs)
    for i in range(16):
      if i & s:
        continue
      j = i ^ s
      a, b = regs[i], regs[j]
      nr[i] = jnp.where(m, a, b.at[p].get(mode="promise_in_bounds"))
      nr[j] = jnp.where(m, a.at[p].get(mode="promise_in_bounds"), b)
    regs = nr
  return regs


@functools.partial(
    pl.kernel,
    mesh=_mesh,
    out_type=jax.ShapeDtypeStruct((_V * _D,), jnp.float32),
    scratch_types=[
        pltpu.VMEM((_D, 128), jnp.float32),
        pltpu.VMEM((_D, 128), jnp.float32),
        pltpu.VMEM((128 * _D,), jnp.float32),
        pltpu.VMEM((128 * _D,), jnp.float32),
        pltpu.SemaphoreType.DMA((2,)),
        pltpu.SemaphoreType.DMA((2,)),
    ],
    compiler_params=pltpu.CompilerParams(
        use_tc_tiling_on_sc=True, needs_layout_passes=False),
)
def _table_to_rowmajor(tt_hbm, lin_hbm, in_t0, in_t1, out_t0, out_t1,
                       isem, osem):
  wid = lax.axis_index("s") * _NC + lax.axis_index("c")
  in_bufs = (in_t0, in_t1)
  out_bufs = (out_t0, out_t1)

  def tile_of(i):
    return wid + i * _NW

  def start_in(vt, b):
    pltpu.async_copy(
        tt_hbm.at[:, pl.ds(vt * 128, 128)], in_bufs[b], isem.at[b])

  def wait_in(vt, b):
    pltpu.make_async_copy(
        tt_hbm.at[:, pl.ds(vt * 128, 128)], in_bufs[b], isem.at[b]).wait()

  def start_out(vt, b):
    pltpu.async_copy(
        out_bufs[b], lin_hbm.at[pl.ds(vt * (128 * _D), 128 * _D)], osem.at[b])

  def wait_out(vt, b):
    pltpu.make_async_copy(
        out_bufs[b], lin_hbm.at[pl.ds(vt * (128 * _D), 128 * _D)],
        osem.at[b]).wait()

  def transpose_tile(b, n_lanes):
    src = in_bufs[b]
    dst = out_bufs[b]

    @plsc.parallel_loop(0, n_lanes // _L, unroll=4)
    def _(g):
      vl0 = g * _L
      for d0 in range(0, _D, _L):
        regs = [src[d0 + i, pl.ds(vl0, _L)] for i in range(_L)]
        out = _bfly16(regs)
        for j in range(_L):
          dst[pl.ds((vl0 + j) * _D + d0, _L)] = out[j]

  n_main = _TPW

  start_in(tile_of(0), 0)
  start_in(tile_of(1), 1)

  def body(i, carry):
    for b in range(2):
      it = i * 2 + b
      vt = tile_of(it)
      wait_in(vt, b)

      @pl.when(it >= 2)
      def _():
        wait_out(tile_of(it - 2), b)

      transpose_tile(b, 128)

      @pl.when(it + 2 < n_main)
      def _():
        start_in(tile_of(it + 2), b)

      start_out(vt, b)
    return carry

  lax.fori_loop(0, n_main // 2, body, 0)
  wait_out(tile_of(n_main - 2), 0)
  wait_out(tile_of(n_main - 1), 1)

  @pl.when(wid < _VT_REST)
  def _():
    vt = _TPW * _NW + wid
    pltpu.sync_copy(
        tt_hbm.at[:, pl.ds(vt * 128, 128)], in_t0.at[:, pl.ds(0, 128)])
    transpose_tile(0, 128)
    pltpu.sync_copy(out_t0, lin_hbm.at[pl.ds(vt * (128 * _D), 128 * _D)])


_BPW = _B // _NW


@functools.partial(
    pl.kernel,
    mesh=_mesh,
    out_type=jax.ShapeDtypeStruct((_S, 8, _NW, 8 * 128), jnp.float32),
    scratch_types=[
        pltpu.VMEM((_BPW, _S), jnp.int32),
        pltpu.VMEM((_S, _BPW), jnp.int32),
        pltpu.VMEM((_V_TAIL, _D), jnp.float32),
        pltpu.VMEM((2, _BPW, _D), jnp.float32),
        pltpu.VMEM((2, 8, 8 * 128), jnp.float32),
        pltpu.SemaphoreType.DMA((2,)),
        pltpu.SemaphoreType.DMA((2,)),
    ],
    compiler_params=pltpu.CompilerParams(
        use_tc_tiling_on_sc=False, needs_layout_passes=False),
)
def _gather_to_tiled(idx_hbm, tlin_hbm, tail_hbm, out_hbm, xbuf, xt,
                     tail_buf, rows, tbuf, gsem, osem):
  wid = lax.axis_index("s") * _NC + lax.axis_index("c")

  pltpu.sync_copy(idx_hbm.at[pl.ds(wid * _BPW, _BPW), :], xbuf)
  pltpu.sync_copy(tail_hbm, tail_buf)

  tail_mask = lax.iota(jnp.int32, 16) >= (_L - _S % _L)

  @plsc.parallel_loop(0, _BPW, unroll=8)
  def _(bb):
    bsplat = jnp.full((16,), bb, jnp.int32)
    for s0 in range(0, _S - _L + 1, _L):
      vals = xbuf[bb, pl.ds(s0, _L)]
      plsc.store_scatter(xt, [_iota16(s0), bsplat], vals)
    vals = xbuf[bb, pl.ds(_S - _L, _L)]
    plsc.store_scatter(xt, [_iota16(_S - _L), bsplat], vals, mask=tail_mask)

  def start_gather(s, b):
    pltpu.async_copy(tlin_hbm.at[xt.at[s]], rows.at[b], gsem.at[b])

  def wait_gather(s, b):
    pltpu.make_async_copy(
        tlin_hbm.at[xt.at[s]], rows.at[b], gsem.at[b]).wait()

  def start_out(s, b):
    pltpu.async_copy(tbuf.at[b], out_hbm.at[s, :, wid], osem.at[b])

  def wait_out(s, b):
    pltpu.make_async_copy(
        tbuf.at[b], out_hbm.at[s, :, wid], osem.at[b]).wait()

  def transpose_rows(b):
    src = rows.at[b]
    dst = tbuf.at[b]

    @plsc.parallel_loop(0, _BPW // _L, unroll=4)
    def _(g):
      vl0 = g * _L
      for d0 in range(0, _D, _L):
        regs = [src[vl0 + i, pl.ds(d0, _L)] for i in range(_L)]
        out = _bfly16(regs)
        for j in range(_L):
          d = d0 + j
          dst[d // 8, pl.ds((d % 8) * 128 + vl0, _L)] = out[j]

  def fixup_tail(s, b):
    dst = tbuf.at[b]

    def fix_group(g, carry):
      b0 = g * _L
      vv = xt[s, pl.ds(b0, _L)]
      m = vv >= _V_MAIN
      tidx = jnp.maximum(vv - _V_MAIN, 0)
      for d in range(_D):
        col = jnp.full((16,), d, jnp.int32)
        fix = plsc.load_gather(tail_buf, [tidx, col])
        cur = dst[d // 8, pl.ds((d % 8) * 128 + b0, _L)]
        dst[d // 8, pl.ds((d % 8) * 128 + b0, _L)] = jnp.where(m, fix, cur)
      return carry

    lax.fori_loop(0, _BPW // _L, fix_group, 0)

  start_gather(0, 0)
  start_gather(1, 1)

  def body(i, carry):
    for b in range(2):
      s = i * 2 + b
      wait_gather(s, b)

      @pl.when(s >= 2)
      def _():
        wait_out(s - 2, b)

      transpose_rows(b)

      needs_fix = jnp.zeros((), jnp.bool_)
      for b0 in range(0, _BPW, _L):
        vv = xt[s, pl.ds(b0, _L)]
        needs_fix = needs_fix | jnp.any(vv >= _V_MAIN)

      @pl.when(needs_fix)
      def _():
        fixup_tail(s, b)

      @pl.when(s + 2 < _S)
      def _():
        start_gather(s + 2, b)

      start_out(s, b)
    return carry

  lax.fori_loop(0, _S // 2, body, 0)
  wait_out(_S - 2, 0)
  wait_out(_S - 1, 1)


def kernel(x, table):
  b, s = x.shape
  idx2d = x.astype(jnp.int32)
  tt = jnp.swapaxes(table, 0, 1)
  tail = lax.slice(table, (_V_MAIN, 0), (_V, _D))
  tlin = _table_to_rowmajor(tt)
  out4 = _gather_to_tiled(idx2d, tlin.reshape(_V, _D), tail)
  out5 = out4.reshape(_S, 8, _NW, 8, 128)
  out = out5.transpose(2, 4, 0, 1, 3).reshape(b, s, _D)
  return out


# speedup vs baseline: 1.1920x; 1.1920x over previous
import functools

import jax
import jax.numpy as jnp
from jax import lax
from jax.experimental import pallas as pl
from jax.experimental.pallas import tpu as pltpu
from jax.experimental.pallas import tpu_sc as plsc

_D = 64
_V = 1000000
_B = 4096
_S = 200
_L = 16

_info = plsc.get_sparse_core_info()
_NC, _NS = _info.num_cores, _info.num_subcores
_NW = _NC * _NS
_VT = (_V + 127) // 128
_VT_FULL = _V // 128
_TPW = _VT_FULL // _NW
_VT_REST = _VT_FULL - _TPW * _NW
_V_MAIN = _VT_FULL * 128
_V_TAIL = _V - _V_MAIN

_mesh = plsc.VectorSubcoreMesh(core_axis_name="c", subcore_axis_name="s")


def _iota16(off=0):
  return lax.iota(jnp.int32, 16) + off


def _bfly16(regs):
  for s in (1, 2, 4, 8):
    m = (lax.iota(jnp.int32, 16) & s) == 0
    p = lax.iota(jnp.int32, 16) ^ s
    nr = list(regs)
    for i in range(16):
      if i & s:
        continue
      j = i ^ s
      a, b = regs[i], regs[j]
      nr[i] = jnp.where(m, a, b.at[p].get(mode="promise_in_bounds"))
      nr[j] = jnp.where(m, a.at[p].get(mode="promise_in_bounds"), b)
    regs = nr
  return regs


@functools.partial(
    pl.kernel,
    mesh=_mesh,
    out_type=jax.ShapeDtypeStruct((_V * _D,), jnp.float32),
    scratch_types=[
        pltpu.VMEM((_D, 128), jnp.float32),
        pltpu.VMEM((_D, 128), jnp.float32),
        pltpu.VMEM((128 * _D,), jnp.float32),
        pltpu.VMEM((128 * _D,), jnp.float32),
        pltpu.SemaphoreType.DMA((2,)),
        pltpu.SemaphoreType.DMA((2,)),
    ],
    compiler_params=pltpu.CompilerParams(
        use_tc_tiling_on_sc=True, needs_layout_passes=False),
)
def _table_to_rowmajor(tt_hbm, lin_hbm, in_t0, in_t1, out_t0, out_t1,
                       isem, osem):
  wid = lax.axis_index("s") * _NC + lax.axis_index("c")
  in_bufs = (in_t0, in_t1)
  out_bufs = (out_t0, out_t1)

  def tile_of(i):
    return wid + i * _NW

  def start_in(vt, b):
    pltpu.async_copy(
        tt_hbm.at[:, pl.ds(vt * 128, 128)], in_bufs[b], isem.at[b])

  def wait_in(vt, b):
    pltpu.make_async_copy(
        tt_hbm.at[:, pl.ds(vt * 128, 128)], in_bufs[b], isem.at[b]).wait()

  def start_out(vt, b):
    pltpu.async_copy(
        out_bufs[b], lin_hbm.at[pl.ds(vt * (128 * _D), 128 * _D)], osem.at[b])

  def wait_out(vt, b):
    pltpu.make_async_copy(
        out_bufs[b], lin_hbm.at[pl.ds(vt * (128 * _D), 128 * _D)],
        osem.at[b]).wait()

  def transpose_tile(b, n_lanes):
    src = in_bufs[b]
    dst = out_bufs[b]

    @plsc.parallel_loop(0, n_lanes // _L, unroll=2)
    def _(g):
      vl0 = g * _L
      for d0 in range(0, _D, _L):
        regs = [src[d0 + i, pl.ds(vl0, _L)] for i in range(_L)]
        out = _bfly16(regs)
        for j in range(_L):
          dst[pl.ds((vl0 + j) * _D + d0, _L)] = out[j]

  n_main = _TPW

  start_in(tile_of(0), 0)
  start_in(tile_of(1), 1)

  def body(i, carry):
    for b in range(2):
      it = i * 2 + b
      vt = tile_of(it)
      wait_in(vt, b)

      @pl.when(it >= 2)
      def _():
        wait_out(tile_of(it - 2), b)

      transpose_tile(b, 128)

      @pl.when(it + 2 < n_main)
      def _():
        start_in(tile_of(it + 2), b)

      start_out(vt, b)
    return carry

  lax.fori_loop(0, n_main // 2, body, 0)
  wait_out(tile_of(n_main - 2), 0)
  wait_out(tile_of(n_main - 1), 1)

  @pl.when(wid < _VT_REST)
  def _():
    vt = _TPW * _NW + wid
    pltpu.sync_copy(
        tt_hbm.at[:, pl.ds(vt * 128, 128)], in_t0.at[:, pl.ds(0, 128)])
    transpose_tile(0, 128)
    pltpu.sync_copy(out_t0, lin_hbm.at[pl.ds(vt * (128 * _D), 128 * _D)])


_BPW = _B // _NW


@functools.partial(
    pl.kernel,
    mesh=_mesh,
    out_type=jax.ShapeDtypeStruct((_S, 8, _NW, 8 * 128), jnp.float32),
    scratch_types=[
        pltpu.VMEM((_BPW, _S), jnp.int32),
        pltpu.VMEM((_S, _BPW), jnp.int32),
        pltpu.VMEM((_V_TAIL, _D), jnp.float32),
        pltpu.VMEM((2, _BPW, _D), jnp.float32),
        pltpu.VMEM((2, 8, 8 * 128), jnp.float32),
        pltpu.SemaphoreType.DMA((2,)),
        pltpu.SemaphoreType.DMA((2,)),
    ],
    compiler_params=pltpu.CompilerParams(
        use_tc_tiling_on_sc=False, needs_layout_passes=False),
)
def _gather_to_tiled(idx_hbm, tlin_hbm, tail_hbm, out_hbm, xbuf, xt,
                     tail_buf, rows, tbuf, gsem, osem):
  wid = lax.axis_index("s") * _NC + lax.axis_index("c")

  pltpu.sync_copy(idx_hbm.at[pl.ds(wid * _BPW, _BPW), :], xbuf)
  pltpu.sync_copy(tail_hbm, tail_buf)

  tail_mask = lax.iota(jnp.int32, 16) >= (_L - _S % _L)

  @plsc.parallel_loop(0, _BPW, unroll=8)
  def _(bb):
    bsplat = jnp.full((16,), bb, jnp.int32)
    for s0 in range(0, _S - _L + 1, _L):
      vals = xbuf[bb, pl.ds(s0, _L)]
      plsc.store_scatter(xt, [_iota16(s0), bsplat], vals)
    vals = xbuf[bb, pl.ds(_S - _L, _L)]
    plsc.store_scatter(xt, [_iota16(_S - _L), bsplat], vals, mask=tail_mask)

  def start_gather(s, b):
    pltpu.async_copy(tlin_hbm.at[xt.at[s]], rows.at[b], gsem.at[b])

  def wait_gather(s, b):
    pltpu.make_async_copy(
        tlin_hbm.at[xt.at[s]], rows.at[b], gsem.at[b]).wait()

  def start_out(s, b):
    pltpu.async_copy(tbuf.at[b], out_hbm.at[s, :, wid], osem.at[b])

  def wait_out(s, b):
    pltpu.make_async_copy(
        tbuf.at[b], out_hbm.at[s, :, wid], osem.at[b]).wait()

  def transpose_rows(b):
    src = rows.at[b]
    dst = tbuf.at[b]

    @plsc.parallel_loop(0, _BPW // _L, unroll=2)
    def _(g):
      vl0 = g * _L
      for d0 in range(0, _D, _L):
        regs = [src[vl0 + i, pl.ds(d0, _L)] for i in range(_L)]
        out = _bfly16(regs)
        for j in range(_L):
          d = d0 + j
          dst[d // 8, pl.ds((d % 8) * 128 + vl0, _L)] = out[j]

  def fixup_tail(s, b):
    dst = tbuf.at[b]

    def fix_group(g, carry):
      b0 = g * _L
      vv = xt[s, pl.ds(b0, _L)]
      m = vv >= _V_MAIN
      tidx = jnp.maximum(vv - _V_MAIN, 0)
      for d in range(_D):
        col = jnp.full((16,), d, jnp.int32)
        fix = plsc.load_gather(tail_buf, [tidx, col])
        cur = dst[d // 8, pl.ds((d % 8) * 128 + b0, _L)]
        dst[d // 8, pl.ds((d % 8) * 128 + b0, _L)] = jnp.where(m, fix, cur)
      return carry

    lax.fori_loop(0, _BPW // _L, fix_group, 0)

  start_gather(0, 0)
  start_gather(1, 1)

  def body(i, carry):
    for b in range(2):
      s = i * 2 + b
      wait_gather(s, b)

      @pl.when(s >= 2)
      def _():
        wait_out(s - 2, b)

      transpose_rows(b)

      needs_fix = jnp.zeros((), jnp.bool_)
      for b0 in range(0, _BPW, _L):
        vv = xt[s, pl.ds(b0, _L)]
        needs_fix = needs_fix | jnp.any(vv >= _V_MAIN)

      @pl.when(needs_fix)
      def _():
        fixup_tail(s, b)

      @pl.when(s + 2 < _S)
      def _():
        start_gather(s + 2, b)

      start_out(s, b)
    return carry

  lax.fori_loop(0, _S // 2, body, 0)
  wait_out(_S - 2, 0)
  wait_out(_S - 1, 1)


def kernel(x, table):
  b, s = x.shape
  idx2d = x.astype(jnp.int32)
  tt = jnp.swapaxes(table, 0, 1)
  tail = lax.slice(table, (_V_MAIN, 0), (_V, _D))
  tlin = _table_to_rowmajor(tt)
  out4 = _gather_to_tiled(idx2d, tlin.reshape(_V, _D), tail)
  out5 = out4.reshape(_S, 8, _NW, 8, 128)
  out = out5.transpose(2, 4, 0, 1, 3).reshape(b, s, _D)
  return out
